# Initial kernel scaffold; baseline (speedup 1.0000x reference)
import jax
import jax.numpy as jnp
from jax.experimental import pallas as pl


def kernel(x, edge_index, edge_weight, p, W0, W_ih, W_hh, b_ih, b_hh, lin_W, lin_b):
    raise NotImplementedError("write your pallas kernel here")


# speedup vs baseline: 12.9596x; 12.9596x over previous
import functools

import jax
import jax.numpy as jnp
from jax import lax
from jax.experimental import pallas as pl
from jax.experimental.pallas import tpu as pltpu
from jax.experimental.pallas import tpu_sc as plsc

N = 10000
F = 128
E = 320000

NC = 2
NS = 16
EPC = E // NC
EPT = EPC // NS
CK = 80
NCHUNK = EPT // CK
NPAD = 10240
DPT = NPAD // NS
RPT = NPAD // NS
RB = 128

_mesh = functools.partial(
    plsc.VectorSubcoreMesh, core_axis_name="c", subcore_axis_name="s"
)


def _score_body(x_ref, P_ref, o_ref):
    s = jnp.dot(x_ref[...], P_ref[...], precision=lax.Precision.HIGHEST)
    o_ref[...] = s[:, 0:1]


def _score_call(x, P0):
    return pl.pallas_call(
        _score_body,
        grid=(10,),
        in_specs=[
            pl.BlockSpec((N // 10, F), lambda i: (i, 0)),
            pl.BlockSpec((F, F), lambda i: (0, 0)),
        ],
        out_specs=pl.BlockSpec((N // 10, 1), lambda i: (i, 0)),
        out_shape=jax.ShapeDtypeStruct((N, 1), jnp.float32),
    )(x, P0)


def _deg_body(dst_hbm, ew_hbm, degp_hbm, idx_v, val_v, zb_v, deg_sh):
    cid = lax.axis_index("c")
    sid = lax.axis_index("s")
    base = cid * EPC + sid * EPT

    for i in range(DPT // 16):
        zb_v[pl.ds(i * 16, 16)] = jnp.zeros((16,), jnp.float32)
    pltpu.sync_copy(zb_v, deg_sh.at[pl.ds(sid * DPT, DPT)])
    plsc.subcore_barrier()

    def chunk(i, carry):
        off = base + i * CK
        pltpu.sync_copy(dst_hbm.at[pl.ds(off, CK)], idx_v)
        pltpu.sync_copy(ew_hbm.at[pl.ds(off, CK)], val_v)
        pltpu.sync_copy(val_v, deg_sh.at[idx_v], add=True)
        return carry

    lax.fori_loop(0, NCHUNK, chunk, 0)
    plsc.subcore_barrier()
    pltpu.sync_copy(
        deg_sh.at[pl.ds(sid * DPT, DPT)],
        degp_hbm.at[cid, pl.ds(sid * DPT, DPT)],
    )


def _deg_call(dst, ew):
    return pl.kernel(
        _deg_body,
        out_type=jax.ShapeDtypeStruct((NC, NPAD), jnp.float32),
        mesh=_mesh(),
        scratch_types=[
            pltpu.VMEM((CK,), jnp.int32),
            pltpu.VMEM((CK,), jnp.float32),
            pltpu.VMEM((DPT,), jnp.float32),
            pltpu.VMEM_SHARED((NPAD,), jnp.float32),
        ],
    )(dst, ew)


def _dinv_body(degp_ref, o_ref):
    d = degp_ref[0] + degp_ref[1] + 1.0
    o_ref[...] = lax.rsqrt(d)


def _dinv_call(degp3):
    return pl.pallas_call(
        _dinv_body,
        in_specs=[pl.BlockSpec((NC, NPAD // F, F), lambda: (0, 0, 0))],
        out_specs=pl.BlockSpec((NPAD // F, F), lambda: (0, 0)),
        out_shape=jax.ShapeDtypeStruct((NPAD // F, F), jnp.float32),
    )(degp3)


def _gru_body(
    x_ref, perm_ref, topv_ref, W0_ref, Wih_ref, Whh_ref, bih_ref, bhh_ref,
    dinv_ref, y_ref, xt_scr
):
    def gather(j, carry):
        xt_scr[pl.ds(j, 1), :] = x_ref[pl.ds(perm_ref[j], 1), :] * topv_ref[j]
        return carry

    lax.fori_loop(0, F, gather, 0)
    xt = xt_scr[...]
    hi = lax.Precision.HIGHEST
    gi = lax.dot_general(xt, Wih_ref[...], (((1,), (1,)), ((), ())), precision=hi) + bih_ref[...]
    gh = lax.dot_general(W0_ref[...], Whh_ref[...], (((1,), (1,)), ((), ())), precision=hi) + bhh_ref[...]
    r = jax.nn.sigmoid(gi[:, :F] + gh[:, :F])
    z = jax.nn.sigmoid(gi[:, F:2 * F] + gh[:, F:2 * F])
    n = jnp.tanh(gi[:, 2 * F:] + r * gh[:, 2 * F:])
    W = (1.0 - z) * n + z * W0_ref[...]
    xw = jnp.dot(x_ref[...], W, precision=hi)
    y_ref[...] = xw * dinv_ref[...]


def _gru_call(x, perm, topv, W0, Wih, Whh, bih2, bhh2, dinv):
    smem = pl.BlockSpec(memory_space=pltpu.SMEM)
    vmem = pl.BlockSpec(memory_space=pltpu.VMEM)
    return pl.pallas_call(
        _gru_body,
        in_specs=[vmem, smem, smem, vmem, vmem, vmem, vmem, vmem, vmem],
        out_specs=vmem,
        out_shape=jax.ShapeDtypeStruct((N, F), jnp.float32),
        scratch_shapes=[pltpu.VMEM((F, F), jnp.float32)],
    )(x, perm, topv, W0, Wih, Whh, bih2, bhh2, dinv)


def _msg_body(
    src_hbm, dst_hbm, ew_hbm, y_hbm, aggp_hbm,
    sidx_v, didx_v, ew_v, rows_v, zb_v, agg_sh, sem
):
    cid = lax.axis_index("c")
    sid = lax.axis_index("s")
    base = cid * EPC + sid * EPT

    def zrow(i, carry):
        for j in range(F // 16):
            zb_v[i, pl.ds(j * 16, 16)] = jnp.zeros((16,), jnp.float32)
        return carry

    lax.fori_loop(0, RB, zrow, 0)
    for j in range(RPT // RB):
        pltpu.sync_copy(zb_v, agg_sh.at[pl.ds(sid * RPT + j * RB, RB)])

    plsc.subcore_barrier()

    def chunk(i, carry):
        off = base + i * CK
        pltpu.sync_copy(src_hbm.at[pl.ds(off, CK)], sidx_v)
        pltpu.sync_copy(dst_hbm.at[pl.ds(off, CK)], didx_v)
        pltpu.sync_copy(ew_hbm.at[pl.ds(off, CK)], ew_v)
        pltpu.async_copy(y_hbm.at[sidx_v], rows_v, sem).wait()

        def scale(g, c2):
            s16 = ew_v[pl.ds(g * 16, 16)]
            r0 = g * 16
            for e in range(16):
                s = s16[e]
                for j in range(F // 16):
                    rows_v[r0 + e, pl.ds(j * 16, 16)] = (
                        rows_v[r0 + e, pl.ds(j * 16, 16)] * s
                    )
            return c2

        lax.fori_loop(0, CK // 16, scale, 0)
        pltpu.sync_copy(rows_v, agg_sh.at[didx_v], add=True)
        return carry

    lax.fori_loop(0, NCHUNK, chunk, 0)
    plsc.subcore_barrier()

    for j in range(RPT // RB):
        r0 = sid * RPT + j * RB
        pltpu.sync_copy(agg_sh.at[pl.ds(r0, RB)], zb_v)
        pltpu.sync_copy(zb_v, aggp_hbm.at[cid, pl.ds(r0, RB)])


def _msg_call(src, dst, ew, y):
    return pl.kernel(
        _msg_body,
        out_type=jax.ShapeDtypeStruct((NC, NPAD, F), jnp.float32),
        mesh=_mesh(),
        scratch_types=[
            pltpu.VMEM((CK,), jnp.int32),
            pltpu.VMEM((CK,), jnp.int32),
            pltpu.VMEM((CK,), jnp.float32),
            pltpu.VMEM((CK, F), jnp.float32),
            pltpu.VMEM((RB, F), jnp.float32),
            pltpu.VMEM_SHARED((NPAD, F), jnp.float32),
            pltpu.SemaphoreType.DMA,
        ],
    )(src, dst, ew, y)


def _final_body(a0_ref, a1_ref, y_ref, dinv_ref, lw_ref, lb_ref, o_ref):
    agg = (a0_ref[0] + a1_ref[0] + y_ref[...]) * dinv_ref[...]
    h = jnp.maximum(agg, 0.0)
    o = jnp.dot(h, lw_ref[...], precision=lax.Precision.HIGHEST)
    o_ref[...] = o[:, 0:1] + lb_ref[0]


def _final_call(aggp, y, dinv, lw, lb):
    B = N // 10
    blk = lambda i: (i, 0)
    return pl.pallas_call(
        _final_body,
        grid=(10,),
        in_specs=[
            pl.BlockSpec((1, B, F), lambda i: (0, i, 0)),
            pl.BlockSpec((1, B, F), lambda i: (1, i, 0)),
            pl.BlockSpec((B, F), blk),
            pl.BlockSpec((B, 1), blk),
            pl.BlockSpec((F, F), lambda i: (0, 0)),
            pl.BlockSpec(memory_space=pltpu.SMEM),
        ],
        out_specs=pl.BlockSpec((B, 1), blk),
        out_shape=jax.ShapeDtypeStruct((N, 1), jnp.float32),
    )(aggp, aggp, y, dinv, lw, lb)


def kernel(x, edge_index, edge_weight, p, W0, W_ih, W_hh, b_ih, b_hh, lin_W, lin_b):
    src = edge_index[0]
    dst = edge_index[1]

    score = jnp.tanh((x @ p) / jnp.linalg.norm(p))
    topv, perm = lax.top_k(score, F)

    degp = _deg_call(dst, edge_weight)
    dinv2d = _dinv_call(degp.reshape(NC, NPAD // F, F))
    dinv = dinv2d.reshape(NPAD)[:N, None]

    y = _gru_call(
        x, perm, topv, W0, W_ih, W_hh,
        b_ih.reshape(1, 3 * F), b_hh.reshape(1, 3 * F), dinv,
    )
    aggp = _msg_call(src, dst, edge_weight, y)
    LW = jnp.zeros((F, F), jnp.float32).at[:, 0].set(lin_W[0])
    return _final_call(aggp, y, dinv, LW, lin_b)


# speedup vs baseline: 31.4994x; 2.4306x over previous
import functools

import jax
import jax.numpy as jnp
from jax import lax
from jax.experimental import pallas as pl
from jax.experimental.pallas import tpu as pltpu
from jax.experimental.pallas import tpu_sc as plsc

N = 10000
F = 128
E = 320000

NC = 2
NS = 16
EPC = E // NC
EPT = EPC // NS
CK = 80
NCHUNK = EPT // CK
BCH = 25
NBLK = NCHUNK // BCH
NPAD = 10240
DPT = NPAD // NS
RPT = NPAD // NS
RB = 128

_mesh = functools.partial(
    plsc.VectorSubcoreMesh, core_axis_name="c", subcore_axis_name="s"
)


def _score_body(x_ref, P_ref, o_ref):
    s = jnp.dot(x_ref[...], P_ref[...], precision=lax.Precision.HIGHEST)
    o_ref[...] = s[:, 0:1]


def _score_call(x, P0):
    return pl.pallas_call(
        _score_body,
        grid=(10,),
        in_specs=[
            pl.BlockSpec((N // 10, F), lambda i: (i, 0)),
            pl.BlockSpec((F, F), lambda i: (0, 0)),
        ],
        out_specs=pl.BlockSpec((N // 10, 1), lambda i: (i, 0)),
        out_shape=jax.ShapeDtypeStruct((N, 1), jnp.float32),
    )(x, P0)


def _deg_body(dst4, ew4, degp_hbm, blk_d, blk_w, zb_v, deg_sh, blk_sem):
    cid = lax.axis_index("c")
    sid = lax.axis_index("s")
    wid = cid * NS + sid

    for i in range(DPT // 16):
        zb_v[pl.ds(i * 16, 16)] = jnp.zeros((16,), jnp.float32)
    pltpu.sync_copy(zb_v, deg_sh.at[pl.ds(sid * DPT, DPT)])
    plsc.subcore_barrier()

    def block(b, carry):
        pltpu.sync_copy(dst4.at[wid, b], blk_d)
        pltpu.sync_copy(ew4.at[wid, b], blk_w)
        descs = [
            pltpu.async_copy(blk_w.at[i], deg_sh.at[blk_d.at[i]], blk_sem,
                             add=True)
            for i in range(BCH)
        ]
        for d in descs:
            d.wait()
        return carry

    lax.fori_loop(0, NBLK, block, 0)
    plsc.subcore_barrier()
    pltpu.sync_copy(
        deg_sh.at[pl.ds(sid * DPT, DPT)],
        degp_hbm.at[cid, pl.ds(sid * DPT, DPT)],
    )


def _deg_call(dst4, ew4):
    return pl.kernel(
        _deg_body,
        out_type=jax.ShapeDtypeStruct((NC, NPAD), jnp.float32),
        mesh=_mesh(),
        scratch_types=[
            pltpu.VMEM((BCH, CK), jnp.int32),
            pltpu.VMEM((BCH, CK), jnp.float32),
            pltpu.VMEM((DPT,), jnp.float32),
            pltpu.VMEM_SHARED((NPAD,), jnp.float32),
            pltpu.SemaphoreType.DMA,
        ],
    )(dst4, ew4)


def _dinv_body(degp_ref, o_ref):
    d = degp_ref[0] + degp_ref[1] + 1.0
    o_ref[...] = lax.rsqrt(d)


def _dinv_call(degp3):
    return pl.pallas_call(
        _dinv_body,
        in_specs=[pl.BlockSpec((NC, NPAD // F, F), lambda: (0, 0, 0))],
        out_specs=pl.BlockSpec((NPAD // F, F), lambda: (0, 0)),
        out_shape=jax.ShapeDtypeStruct((NPAD // F, F), jnp.float32),
    )(degp3)


def _gru_body(
    x_ref, perm_ref, topv_ref, W0_ref, Wih_ref, Whh_ref, bih_ref, bhh_ref,
    dinv_ref, y_ref, xt_scr
):
    def gather(j, carry):
        xt_scr[pl.ds(j, 1), :] = x_ref[pl.ds(perm_ref[j], 1), :] * topv_ref[j]
        return carry

    lax.fori_loop(0, F, gather, 0)
    xt = xt_scr[...]
    hi = lax.Precision.HIGHEST
    gi = lax.dot_general(xt, Wih_ref[...], (((1,), (1,)), ((), ())), precision=hi) + bih_ref[...]
    gh = lax.dot_general(W0_ref[...], Whh_ref[...], (((1,), (1,)), ((), ())), precision=hi) + bhh_ref[...]
    r = jax.nn.sigmoid(gi[:, :F] + gh[:, :F])
    z = jax.nn.sigmoid(gi[:, F:2 * F] + gh[:, F:2 * F])
    n = jnp.tanh(gi[:, 2 * F:] + r * gh[:, 2 * F:])
    W = (1.0 - z) * n + z * W0_ref[...]
    xw = jnp.dot(x_ref[...], W, precision=hi)
    y_ref[...] = xw * dinv_ref[...]


def _gru_call(x, perm, topv, W0, Wih, Whh, bih2, bhh2, dinv):
    smem = pl.BlockSpec(memory_space=pltpu.SMEM)
    vmem = pl.BlockSpec(memory_space=pltpu.VMEM)
    return pl.pallas_call(
        _gru_body,
        in_specs=[vmem, smem, smem, vmem, vmem, vmem, vmem, vmem, vmem],
        out_specs=vmem,
        out_shape=jax.ShapeDtypeStruct((N, F), jnp.float32),
        scratch_shapes=[pltpu.VMEM((F, F), jnp.float32)],
    )(x, perm, topv, W0, Wih, Whh, bih2, bhh2, dinv)


def _msg_body(
    src4, dst4, ew4, y_hbm, aggp_hbm,
    blk_s, blk_d, blk_w, rows_a, rows_b, agg_sh, gsem_a, gsem_b
):
    cid = lax.axis_index("c")
    sid = lax.axis_index("s")
    wid = cid * NS + sid

    def zrow(i, carry):
        for j in range(F // 16):
            rows_a[i, pl.ds(j * 16, 16)] = jnp.zeros((16,), jnp.float32)
        return carry

    lax.fori_loop(0, CK, zrow, 0)
    for j in range(RPT // CK):
        pltpu.sync_copy(rows_a, agg_sh.at[pl.ds(sid * RPT + j * CK, CK)])
    plsc.subcore_barrier()

    def gstart(buf, sem, c):
        pltpu.async_copy(y_hbm.at[blk_s.at[c]], buf, sem)

    def gwait(buf, sem, c):
        pltpu.make_async_copy(y_hbm.at[blk_s.at[c]], buf, sem).wait()

    def scale(buf, c):
        def grp(g, c2):
            s16 = blk_w[c, pl.ds(g * 16, 16)]
            r0 = g * 16
            for e in range(16):
                s = s16[e]
                for j in range(F // 16):
                    buf[r0 + e, pl.ds(j * 16, 16)] = (
                        buf[r0 + e, pl.ds(j * 16, 16)] * s
                    )
            return c2

        lax.fori_loop(0, CK // 16, grp, 0)

    def scatter(buf, c):
        pltpu.sync_copy(buf, agg_sh.at[blk_d.at[c]], add=True)

    def block(b, carry):
        pltpu.sync_copy(src4.at[wid, b], blk_s)
        pltpu.sync_copy(dst4.at[wid, b], blk_d)
        pltpu.sync_copy(ew4.at[wid, b], blk_w)
        gstart(rows_a, gsem_a, 0)

        def pair(i, c2):
            c0 = 2 * i
            gstart(rows_b, gsem_b, c0 + 1)
            gwait(rows_a, gsem_a, c0)
            scale(rows_a, c0)
            scatter(rows_a, c0)
            gstart(rows_a, gsem_a, c0 + 2)
            gwait(rows_b, gsem_b, c0 + 1)
            scale(rows_b, c0 + 1)
            scatter(rows_b, c0 + 1)
            return c2

        lax.fori_loop(0, (BCH - 1) // 2, pair, 0)
        gwait(rows_a, gsem_a, BCH - 1)
        scale(rows_a, BCH - 1)
        scatter(rows_a, BCH - 1)
        return carry

    lax.fori_loop(0, NBLK, block, 0)
    plsc.subcore_barrier()

    for j in range(RPT // CK):
        r0 = sid * RPT + j * CK
        pltpu.sync_copy(agg_sh.at[pl.ds(r0, CK)], aggp_hbm.at[cid, pl.ds(r0, CK)])


def _msg_call(src4, dst4, ew4, y):
    return pl.kernel(
        _msg_body,
        out_type=jax.ShapeDtypeStruct((NC, NPAD, F), jnp.float32),
        mesh=_mesh(),
        scratch_types=[
            pltpu.VMEM((BCH, CK), jnp.int32),
            pltpu.VMEM((BCH, CK), jnp.int32),
            pltpu.VMEM((BCH, CK), jnp.float32),
            pltpu.VMEM((CK, F), jnp.float32),
            pltpu.VMEM((CK, F), jnp.float32),
            pltpu.VMEM_SHARED((NPAD, F), jnp.float32),
            pltpu.SemaphoreType.DMA,
            pltpu.SemaphoreType.DMA,
        ],
    )(src4, dst4, ew4, y)


def _final_body(a0_ref, a1_ref, y_ref, dinv_ref, lw_ref, lb_ref, o_ref):
    agg = (a0_ref[0] + a1_ref[0] + y_ref[...]) * dinv_ref[...]
    h = jnp.maximum(agg, 0.0)
    o = jnp.dot(h, lw_ref[...], precision=lax.Precision.HIGHEST)
    o_ref[...] = o[:, 0:1] + lb_ref[0]


def _final_call(aggp, y, dinv, lw, lb):
    B = N // 10
    blk = lambda i: (i, 0)
    return pl.pallas_call(
        _final_body,
        grid=(10,),
        in_specs=[
            pl.BlockSpec((1, B, F), lambda i: (0, i, 0)),
            pl.BlockSpec((1, B, F), lambda i: (1, i, 0)),
            pl.BlockSpec((B, F), blk),
            pl.BlockSpec((B, 1), blk),
            pl.BlockSpec((F, F), lambda i: (0, 0)),
            pl.BlockSpec(memory_space=pltpu.SMEM),
        ],
        out_specs=pl.BlockSpec((B, 1), blk),
        out_shape=jax.ShapeDtypeStruct((N, 1), jnp.float32),
    )(aggp, aggp, y, dinv, lw, lb)


def kernel(x, edge_index, edge_weight, p, W0, W_ih, W_hh, b_ih, b_hh, lin_W, lin_b):
    src4 = edge_index[0].reshape(NC * NS, NBLK, BCH, CK)
    dst4 = edge_index[1].reshape(NC * NS, NBLK, BCH, CK)
    ew4 = edge_weight.reshape(NC * NS, NBLK, BCH, CK)

    score = jnp.tanh((x @ p) / jnp.linalg.norm(p))
    topv, perm = lax.top_k(score, F)

    degp = _deg_call(dst4, ew4)
    dinv2d = _dinv_call(degp.reshape(NC, NPAD // F, F))
    dinv = dinv2d.reshape(NPAD)[:N, None]

    y = _gru_call(
        x, perm, topv, W0, W_ih, W_hh,
        b_ih.reshape(1, 3 * F), b_hh.reshape(1, 3 * F), dinv,
    )
    aggp = _msg_call(src4, dst4, ew4, y)
    LW = jnp.zeros((F, F), jnp.float32).at[:, 0].set(lin_W[0])
    return _final_call(aggp, y, dinv, LW, lin_b)
